# baseline (speedup 1.0000x reference)
import functools

import jax
import jax.numpy as jnp
from jax.experimental import pallas as pl


def _silu(x):
    return x * jax.nn.sigmoid(x)


def _fused_body(x_ref, e_ref, ids_ref,
                wm0_ref, bm0_ref, wg0_ref, bg0_ref,
                wm1_ref, bm1_ref, wg1_ref, bg1_ref,
                wm2_ref, bm2_ref, wg2_ref, bg2_ref,
                out_atom_ref, out_seg_ref, *, num_graphs):
    f32 = jnp.float32
    bf16 = jnp.bfloat16
    x = x_ref[...]
    h = jnp.dot(x, wm0_ref[...], preferred_element_type=f32) + bm0_ref[...]
    g = jnp.dot(x, wg0_ref[...], preferred_element_type=f32) + bg0_ref[...]
    h = _silu(h).astype(bf16)
    g = _silu(g).astype(bf16)
    h = _silu(jnp.dot(h, wm1_ref[...], preferred_element_type=f32) + bm1_ref[...])
    g = _silu(jnp.dot(g, wg1_ref[...], preferred_element_type=f32) + bg1_ref[...])
    h2 = jnp.dot(h.astype(bf16), wm2_ref[...], preferred_element_type=f32) + bm2_ref[...]
    g2 = jax.nn.sigmoid(jnp.dot(g.astype(bf16), wg2_ref[...], preferred_element_type=f32) + bg2_ref[...])
    scaled = e_ref[...] + h2 * g2
    out_atom_ref[...] = scaled

    ids = ids_ref[...]
    blk = ids.shape[0]
    iota = jax.lax.broadcasted_iota(jnp.int32, (blk, num_graphs), 1)
    onehot = (ids == iota).astype(f32)
    partial = jax.lax.dot_general(scaled, onehot,
                                  (((0,), (0,)), ((), ())),
                                  preferred_element_type=f32)

    @pl.when(pl.program_id(0) == 0)
    def _init():
        out_seg_ref[...] = jnp.zeros_like(out_seg_ref)

    out_seg_ref[...] += partial


def kernel(node_features, elemental_energies, batch, lattice,
           Wm0, bm0, Wg0, bg0, Wm1, bm1, Wg1, bg1, Wm2, bm2, Wg2, bg2):
    scale = 1.0
    n, d = node_features.shape
    num_graphs = lattice.shape[0]

    blk = 2000
    n_pad = -(-n // blk) * blk
    if n_pad != n:
        node_features = jnp.pad(node_features, ((0, n_pad - n), (0, 0)))
        elemental_energies = jnp.pad(elemental_energies, (0, n_pad - n))
        batch = jnp.pad(batch, (0, n_pad - n), constant_values=-1)
    grid = n_pad // blk

    bf16 = jnp.bfloat16
    node_features = node_features.astype(bf16)
    Wm0, Wg0, Wm1, Wg1, Wm2, Wg2 = (w.astype(bf16)
                                    for w in (Wm0, Wg0, Wm1, Wg1, Wm2, Wg2))

    e2 = elemental_energies.reshape(n_pad, 1)
    ids2 = batch.reshape(n_pad, 1)
    b_m0 = bm0.reshape(1, d)
    b_g0 = bg0.reshape(1, d)
    b_m1 = bm1.reshape(1, d)
    b_g1 = bg1.reshape(1, d)
    b_m2 = bm2.reshape(1, 1)
    b_g2 = bg2.reshape(1, 1)

    row_spec = pl.BlockSpec((blk, d), lambda i: (i, 0))
    col_spec = pl.BlockSpec((blk, 1), lambda i: (i, 0))
    full = lambda shp: pl.BlockSpec(shp, lambda i: (0,) * len(shp))

    out_atom, out_seg = pl.pallas_call(
        functools.partial(_fused_body, num_graphs=num_graphs),
        grid=(grid,),
        in_specs=[
            row_spec,
            col_spec,
            col_spec,
            full((d, d)), full((1, d)),
            full((d, d)), full((1, d)),
            full((d, d)), full((1, d)),
            full((d, d)), full((1, d)),
            full((d, 1)), full((1, 1)),
            full((d, 1)), full((1, 1)),
        ],
        out_specs=[
            pl.BlockSpec((blk, 1), lambda i: (i, 0)),
            pl.BlockSpec((1, num_graphs), lambda i: (0, 0)),
        ],
        out_shape=[
            jax.ShapeDtypeStruct((n_pad, 1), jnp.float32),
            jax.ShapeDtypeStruct((1, num_graphs), jnp.float32),
        ],
    )(node_features, e2, ids2,
      Wm0, b_m0, Wg0, b_g0, Wm1, b_m1, Wg1, b_g1, Wm2, b_m2, Wg2, b_g2)

    scaled_atomic = out_atom.reshape(n_pad)[:n]
    scaled_total = out_seg.reshape(num_graphs)
    total = scale * scaled_total
    return (scaled_atomic, scaled_total, total)


# speedup vs baseline: 1.4068x; 1.4068x over previous
import functools

import jax
import jax.numpy as jnp
from jax.experimental import pallas as pl


def _sigmoid(x):
    return 0.5 * jnp.tanh(0.5 * x) + 0.5


def _silu(x):
    t = 0.5 * x
    return t * jnp.tanh(t) + t


def _fused_body(x_ref, e_ref, ids_ref,
                wm0_ref, bm0_ref, wg0_ref, bg0_ref,
                wm1_ref, bm1_ref, wg1_ref, bg1_ref,
                wm2_ref, bm2_ref, wg2_ref, bg2_ref,
                out_atom_ref, out_seg_ref, *, num_graphs):
    f32 = jnp.float32
    bf16 = jnp.bfloat16
    x = x_ref[...]
    h = jnp.dot(x, wm0_ref[...], preferred_element_type=f32) + bm0_ref[...]
    g = jnp.dot(x, wg0_ref[...], preferred_element_type=f32) + bg0_ref[...]
    h = _silu(h)
    g = _silu(g)
    h = _silu(jnp.dot(h, wm1_ref[...], preferred_element_type=f32) + bm1_ref[...])
    g = _silu(jnp.dot(g, wg1_ref[...], preferred_element_type=f32) + bg1_ref[...])
    h2 = jnp.dot(h, wm2_ref[...], preferred_element_type=f32) + bm2_ref[...]
    g2 = _sigmoid(jnp.dot(g, wg2_ref[...], preferred_element_type=f32) + bg2_ref[...])
    scaled = e_ref[...] + h2 * g2
    out_atom_ref[...] = scaled

    ids = ids_ref[...]
    blk = ids.shape[0]
    iota = jax.lax.broadcasted_iota(jnp.int32, (blk, num_graphs), 1)
    onehot = (ids == iota).astype(f32)
    partial = jax.lax.dot_general(scaled, onehot,
                                  (((0,), (0,)), ((), ())),
                                  preferred_element_type=f32)

    @pl.when(pl.program_id(0) == 0)
    def _init():
        out_seg_ref[...] = jnp.zeros_like(out_seg_ref)

    out_seg_ref[...] += partial


def kernel(node_features, elemental_energies, batch, lattice,
           Wm0, bm0, Wg0, bg0, Wm1, bm1, Wg1, bg1, Wm2, bm2, Wg2, bg2):
    scale = 1.0
    n, d = node_features.shape
    num_graphs = lattice.shape[0]

    blk = 2000
    n_pad = -(-n // blk) * blk
    if n_pad != n:
        node_features = jnp.pad(node_features, ((0, n_pad - n), (0, 0)))
        elemental_energies = jnp.pad(elemental_energies, (0, n_pad - n))
        batch = jnp.pad(batch, (0, n_pad - n), constant_values=-1)
    grid = n_pad // blk


    e2 = elemental_energies.reshape(n_pad, 1)
    ids2 = batch.reshape(n_pad, 1)
    b_m0 = bm0.reshape(1, d)
    b_g0 = bg0.reshape(1, d)
    b_m1 = bm1.reshape(1, d)
    b_g1 = bg1.reshape(1, d)
    b_m2 = bm2.reshape(1, 1)
    b_g2 = bg2.reshape(1, 1)

    row_spec = pl.BlockSpec((blk, d), lambda i: (i, 0))
    col_spec = pl.BlockSpec((blk, 1), lambda i: (i, 0))
    full = lambda shp: pl.BlockSpec(shp, lambda i: (0,) * len(shp))

    out_atom, out_seg = pl.pallas_call(
        functools.partial(_fused_body, num_graphs=num_graphs),
        grid=(grid,),
        in_specs=[
            row_spec,
            col_spec,
            col_spec,
            full((d, d)), full((1, d)),
            full((d, d)), full((1, d)),
            full((d, d)), full((1, d)),
            full((d, d)), full((1, d)),
            full((d, 1)), full((1, 1)),
            full((d, 1)), full((1, 1)),
        ],
        out_specs=[
            pl.BlockSpec((blk, 1), lambda i: (i, 0)),
            pl.BlockSpec((1, num_graphs), lambda i: (0, 0)),
        ],
        out_shape=[
            jax.ShapeDtypeStruct((n_pad, 1), jnp.float32),
            jax.ShapeDtypeStruct((1, num_graphs), jnp.float32),
        ],
    )(node_features, e2, ids2,
      Wm0, b_m0, Wg0, b_g0, Wm1, b_m1, Wg1, b_g1, Wm2, b_m2, Wg2, b_g2)

    scaled_atomic = out_atom.reshape(n_pad)[:n]
    scaled_total = out_seg.reshape(num_graphs)
    total = scale * scaled_total
    return (scaled_atomic, scaled_total, total)
